# baseline (speedup 1.0000x reference)
import functools

import numpy as np
import jax
import jax.numpy as jnp
from jax import lax
from jax.experimental import pallas as pl
from jax.experimental.pallas import tpu as pltpu, tpu_sc as plsc

_N = 100000
_MAX_DEG = 10
_COUNTS = [10000] + [9000] * _MAX_DEG
_STARTS = np.cumsum([0] + _COUNTS[:-1]).tolist()
_ABASE = [0] + np.cumsum([9000 * d for d in range(1, _MAX_DEG + 1)]).tolist()
_NC, _NS = 2, 16
_NW = _NC * _NS
_DEG_ROWS = 9000


def _divisors(n):
    return sorted({d for i in range(1, int(n ** 0.5) + 1) if n % i == 0
                   for d in (i, n // i)}, reverse=True)


def _pick_chunk(d, gmax):
    for c in _divisors(_DEG_ROWS):
        if c % 8 == 0 and c * d <= gmax:
            return c
    raise ValueError(d)


def _worker_id():
    return lax.axis_index("s") * _NC + lax.axis_index("c")


def _sc_mesh():
    return plsc.VectorSubcoreMesh(core_axis_name="c", subcore_axis_name="s")


def _conv_plan(gmax):
    return [(d, _pick_chunk(d, gmax), _DEG_ROWS // _pick_chunk(d, gmax))
            for d in range(1, _MAX_DEG + 1)]


def _chunk_loop(n_chunks, body):
    wid = _worker_id()
    n_full = n_chunks // _NW
    rem = n_chunks % _NW
    n_mine = n_full + jnp.where(wid < rem, 1, 0)

    def step(k, carry):
        body(wid + k * _NW)
        return carry

    lax.fori_loop(0, n_mine, step, 0)


def _pipelined_loop(n_chunks, issue, consume):
    wid = _worker_id()
    n_full = n_chunks // _NW
    rem = n_chunks % _NW
    m = n_full + jnp.where(wid < rem, 1, 0)

    @pl.when(m > 0)
    def _():
        issue(wid, 0)

    def pair(k2, carry):
        ka = 2 * k2
        kb = ka + 1

        @pl.when(ka < m)
        def _():
            @pl.when(kb < m)
            def _():
                issue(wid + kb * _NW, 1)
            consume(wid + ka * _NW, 0, ka)

        @pl.when(kb < m)
        def _():
            @pl.when(kb + 1 < m)
            def _():
                issue(wid + (kb + 1) * _NW, 0)
            consume(wid + kb * _NW, 1, kb)

        return carry

    lax.fori_loop(0, (m + 1) // 2, pair, 0)
    return m


def _make_conv_sc(D, gmax):
    plan = _conv_plan(gmax)
    g_rows = max(c * d for d, c, _ in plan)
    acc_rows = max(c for d, c, _ in plan if d > 1)

    def body(table, *rest):
        adjs = rest[:_MAX_DEG]
        out = rest[_MAX_DEG]
        rest = rest[_MAX_DEG + 1:]
        idx_v = rest[0:2]
        rows_v = rest[2:4]
        acc_v = rest[4:6]
        gsem = rest[6:8]
        wsem = rest[8:10]

        for d, C, n_chunks in plan:
            G = C * d
            adj = adjs[d - 1]

            def issue(ci, p, d=d, C=C, G=G, adj=adj):
                pltpu.sync_copy(adj.at[pl.ds(ci * G, G)],
                                idx_v[p].at[pl.ds(0, G)])
                pltpu.async_copy(table.at[idx_v[p].at[pl.ds(0, G)]],
                                 rows_v[p].at[pl.ds(0, G)], gsem[p])

            def consume(ci, p, k, d=d, C=C, G=G):
                out0 = (d - 1) * _DEG_ROWS + ci * C
                pltpu.make_async_copy(table.at[idx_v[p].at[pl.ds(0, G)]],
                                      rows_v[p].at[pl.ds(0, G)],
                                      gsem[p]).wait()
                if d == 1:
                    pltpu.sync_copy(rows_v[p].at[pl.ds(0, C)],
                                    out.at[pl.ds(out0, C)])
                    return

                @pl.when(k >= 2)
                def _():
                    pltpu.make_async_copy(acc_v[p].at[pl.ds(0, C)],
                                          out.at[pl.ds(0, C)], wsem[p]).wait()

                def red(r, carry):
                    for c in range(D // 16):
                        sl = pl.ds(c * 16, 16)
                        s = rows_v[p][r * d, sl]
                        for j in range(1, d):
                            s = s + rows_v[p][r * d + j, sl]
                        acc_v[p][r, sl] = s
                    return carry
                lax.fori_loop(0, C, red, 0)
                pltpu.async_copy(acc_v[p].at[pl.ds(0, C)],
                                 out.at[pl.ds(out0, C)], wsem[p])

            m = _pipelined_loop(n_chunks, issue, consume)
            if d > 1:
                for p in range(2):
                    @pl.when(m > p)
                    def _(p=p, C=C):
                        pltpu.make_async_copy(
                            acc_v[p].at[pl.ds(0, C)],
                            out.at[pl.ds(0, C)], wsem[p]).wait()

    return pl.kernel(
        body,
        out_type=jax.ShapeDtypeStruct((_MAX_DEG * _DEG_ROWS, D), jnp.float32),
        mesh=_sc_mesh(),
        compiler_params=pltpu.CompilerParams(use_tc_tiling_on_sc=(D == 128)),
        scratch_types=[
            pltpu.VMEM((g_rows,), jnp.int32),
            pltpu.VMEM((g_rows,), jnp.int32),
            pltpu.VMEM((g_rows, D), jnp.float32),
            pltpu.VMEM((g_rows, D), jnp.float32),
            pltpu.VMEM((acc_rows, D), jnp.float32),
            pltpu.VMEM((acc_rows, D), jnp.float32),
            pltpu.SemaphoreType.DMA,
            pltpu.SemaphoreType.DMA,
            pltpu.SemaphoreType.DMA,
            pltpu.SemaphoreType.DMA,
        ],
    )


def _make_pool_sc(D, gmax):
    cmax = 120 if D == 128 else 360
    plan = [(d, min(c, cmax), _DEG_ROWS // min(c, cmax))
            for d, c, _ in _conv_plan(gmax)]
    c0 = 200
    n0 = _COUNTS[0] // c0
    g_rows = max(max(c * d for d, c, _ in plan), c0)
    acc_rows = max(c for d, c, _ in plan)

    def body(table, *rest):
        adjs = rest[:_MAX_DEG]
        out = rest[_MAX_DEG]
        rest = rest[_MAX_DEG + 1:]
        idx_v = rest[0:2]
        rows_v = rest[2:4]
        acc_v = rest[4:6]
        gsem = rest[6:8]
        wsem = rest[8:10]

        def copy0(ci):
            row0 = ci * c0
            pltpu.sync_copy(table.at[pl.ds(row0, c0)],
                            rows_v[0].at[pl.ds(0, c0)])
            pltpu.sync_copy(rows_v[0].at[pl.ds(0, c0)],
                            out.at[pl.ds(row0, c0)])

        _chunk_loop(n0, copy0)

        for d, C, n_chunks in plan:
            G = C * d
            adj = adjs[d - 1]
            start = _STARTS[d]

            def issue(ci, p, d=d, C=C, G=G, adj=adj, start=start):
                pltpu.sync_copy(adj.at[pl.ds(ci * G, G)],
                                idx_v[p].at[pl.ds(0, G)])
                pltpu.async_copy(table.at[idx_v[p].at[pl.ds(0, G)]],
                                 rows_v[p].at[pl.ds(0, G)], gsem[p])

            def consume(ci, p, k, d=d, C=C, G=G, start=start):
                row0 = start + ci * C

                @pl.when(k >= 2)
                def _():
                    pltpu.make_async_copy(acc_v[p].at[pl.ds(0, C)],
                                          out.at[pl.ds(0, C)], wsem[p]).wait()

                pltpu.sync_copy(table.at[pl.ds(row0, C)],
                                acc_v[p].at[pl.ds(0, C)])
                pltpu.make_async_copy(table.at[idx_v[p].at[pl.ds(0, G)]],
                                      rows_v[p].at[pl.ds(0, G)],
                                      gsem[p]).wait()

                def red(r, carry):
                    for c in range(D // 16):
                        sl = pl.ds(c * 16, 16)
                        m = acc_v[p][r, sl]
                        for j in range(d):
                            m = jnp.maximum(m, rows_v[p][r * d + j, sl])
                        acc_v[p][r, sl] = m
                    return carry
                lax.fori_loop(0, C, red, 0)
                pltpu.async_copy(acc_v[p].at[pl.ds(0, C)],
                                 out.at[pl.ds(row0, C)], wsem[p])

            m = _pipelined_loop(n_chunks, issue, consume)
            for p in range(2):
                @pl.when(m > p)
                def _(p=p, C=C):
                    pltpu.make_async_copy(
                        acc_v[p].at[pl.ds(0, C)],
                        out.at[pl.ds(0, C)], wsem[p]).wait()

    return pl.kernel(
        body,
        out_type=jax.ShapeDtypeStruct((_N, D), jnp.float32),
        mesh=_sc_mesh(),
        compiler_params=pltpu.CompilerParams(use_tc_tiling_on_sc=(D == 128)),
        scratch_types=[
            pltpu.VMEM((g_rows,), jnp.int32),
            pltpu.VMEM((g_rows,), jnp.int32),
            pltpu.VMEM((g_rows, D), jnp.float32),
            pltpu.VMEM((g_rows, D), jnp.float32),
            pltpu.VMEM((acc_rows, D), jnp.float32),
            pltpu.VMEM((acc_rows, D), jnp.float32),
            pltpu.SemaphoreType.DMA,
            pltpu.SemaphoreType.DMA,
            pltpu.SemaphoreType.DMA,
            pltpu.SemaphoreType.DMA,
        ],
    )


_TC_B = 1000
_TC_G = _N // _TC_B


def _deg_of_block(i):
    return jnp.where(i < 10, 0, (i - 10) // 9 + 1)


def _make_gc_tc(din, dout):
    def body(x_ref, rel_ref, wr_ref, ws_ref, b_ref, sc_ref, sh_ref, out_ref):
        z = (jnp.dot(x_ref[...], ws_ref[0],
                     preferred_element_type=jnp.float32)
             + jnp.dot(rel_ref[...], wr_ref[0],
                       preferred_element_type=jnp.float32)
             + b_ref[0])
        out_ref[...] = jnp.tanh(z) * sc_ref[...] + sh_ref[...]

    return pl.pallas_call(
        body,
        grid=(_TC_G,),
        in_specs=[
            pl.BlockSpec((_TC_B, din), lambda i: (i, 0)),
            pl.BlockSpec((_TC_B, din), lambda i: (jnp.maximum(i - 10, 0), 0)),
            pl.BlockSpec((1, din, dout), lambda i: (_deg_of_block(i), 0, 0)),
            pl.BlockSpec((1, din, dout), lambda i: (_deg_of_block(i), 0, 0)),
            pl.BlockSpec((1, 1, dout), lambda i: (_deg_of_block(i), 0, 0)),
            pl.BlockSpec((1, dout), lambda i: (0, 0)),
            pl.BlockSpec((1, dout), lambda i: (0, 0)),
        ],
        out_specs=pl.BlockSpec((_TC_B, dout), lambda i: (i, 0)),
        out_shape=jax.ShapeDtypeStruct((_N, dout), jnp.float32),
    )


def _make_final_tc():
    def body(x_ref, w1_ref, b1_ref, sc_ref, sh_ref, m_ref, wreg_ref, breg_ref,
             out_ref, sum_ref, max_ref):
        i = pl.program_id(0)
        h = jnp.tanh(jnp.dot(x_ref[...], w1_ref[...],
                             preferred_element_type=jnp.float32) + b1_ref[...])
        h = h * sc_ref[...] + sh_ref[...]
        rows = i * _TC_B + jax.lax.broadcasted_iota(jnp.int32, (_TC_B, 1), 0)
        sums, maxs = [], []
        for b in range(3):
            lo = jnp.int32(0) if b == 0 else m_ref[0, b - 1]
            hi = jnp.int32(_N) if b == 2 else m_ref[0, b]
            msk = (rows >= lo) & (rows < hi)
            sums.append(jnp.sum(jnp.where(msk, h, 0.0), axis=0, keepdims=True))
            maxs.append(jnp.max(jnp.where(msk, h, -jnp.inf), axis=0,
                                keepdims=True))
        s = jnp.concatenate(sums, axis=0)
        mx = jnp.concatenate(maxs, axis=0)

        @pl.when(i == 0)
        def _():
            sum_ref[...] = s
            max_ref[...] = mx

        @pl.when(i > 0)
        def _():
            sum_ref[...] = sum_ref[...] + s
            max_ref[...] = jnp.maximum(max_ref[...], mx)

        @pl.when(i == _TC_G - 1)
        def _():
            mol = jnp.tanh(jnp.concatenate([sum_ref[...], max_ref[...]],
                                           axis=1))
            out_ref[...] = (jnp.dot(mol, wreg_ref[...],
                                    preferred_element_type=jnp.float32)
                            + breg_ref[...])

    return pl.pallas_call(
        body,
        grid=(_TC_G,),
        in_specs=[
            pl.BlockSpec((_TC_B, 128), lambda i: (i, 0)),
            pl.BlockSpec((128, 256), lambda i: (0, 0)),
            pl.BlockSpec((1, 256), lambda i: (0, 0)),
            pl.BlockSpec((1, 256), lambda i: (0, 0)),
            pl.BlockSpec((1, 256), lambda i: (0, 0)),
            pl.BlockSpec((1, 2), lambda i: (0, 0)),
            pl.BlockSpec((512, 1), lambda i: (0, 0)),
            pl.BlockSpec((1, 1), lambda i: (0, 0)),
        ],
        out_specs=pl.BlockSpec((3, 1), lambda i: (0, 0)),
        out_shape=jax.ShapeDtypeStruct((3, 1), jnp.float32),
        scratch_shapes=[
            pltpu.VMEM((3, 256), jnp.float32),
            pltpu.VMEM((3, 256), jnp.float32),
        ],
    )


def _bn_affine(p, eps=1e-3):
    scale = p["gamma"] / jnp.sqrt(p["var"] + eps)
    shift = p["beta"] - p["mean"] * scale
    return scale[None, :], shift[None, :]


def _gc_weights(gc, din, dout):
    wr = jnp.stack([jnp.zeros((din, dout), jnp.float32)]
                   + [gc["W"][2 * (d - 1)] for d in range(1, _MAX_DEG + 1)])
    ws = jnp.stack([gc["W"][2 * _MAX_DEG]]
                   + [gc["W"][2 * d - 1] for d in range(1, _MAX_DEG + 1)])
    b = jnp.stack([gc["b"][2 * _MAX_DEG]]
                  + [gc["b"][2 * (d - 1)] + gc["b"][2 * d - 1]
                     for d in range(1, _MAX_DEG + 1)])
    return wr, ws, b[:, None, :]


def kernel(atom_features, params, deg_slice, membership, deg_adj_1, deg_adj_2,
           deg_adj_3, deg_adj_4, deg_adj_5, deg_adj_6, deg_adj_7, deg_adj_8,
           deg_adj_9, deg_adj_10):
    del deg_slice
    adjf = [a.reshape(-1) for a in
            (deg_adj_1, deg_adj_2, deg_adj_3, deg_adj_4, deg_adj_5, deg_adj_6,
             deg_adj_7, deg_adj_8, deg_adj_9, deg_adj_10)]
    bnd = jnp.stack([jnp.sum(membership < 1), jnp.sum(membership < 2)])
    bnd = bnd.astype(jnp.int32)[None, :]
    p = params

    wr1, ws1, b1 = _gc_weights(p["gc1"], 128, 64)
    sc1, sh1 = _bn_affine(p["bn1"])
    wr2, ws2, b2 = _gc_weights(p["gc2"], 64, 128)
    sc2, sh2 = _bn_affine(p["bn2"])
    sc3, sh3 = _bn_affine(p["bn3"])

    rel1 = _make_conv_sc(128, 360)(atom_features, *adjf)
    h1 = _make_gc_tc(128, 64)(atom_features, rel1, wr1, ws1, b1, sc1, sh1)
    hp1 = _make_pool_sc(64, 384)(h1, *adjf)
    rel2 = _make_conv_sc(64, 384)(hp1, *adjf)
    h2 = _make_gc_tc(64, 128)(hp1, rel2, wr2, ws2, b2, sc2, sh2)
    hp2 = _make_pool_sc(128, 360)(h2, *adjf)

    out = _make_final_tc()(
        hp2, p["dense1"]["W"], p["dense1"]["b"][None, :], sc3, sh3,
        bnd, p["regress"]["W"], p["regress"]["b"][None, :])
    return out


# speedup vs baseline: 1.0046x; 1.0046x over previous
import functools

import numpy as np
import jax
import jax.numpy as jnp
from jax import lax
from jax.experimental import pallas as pl
from jax.experimental.pallas import tpu as pltpu, tpu_sc as plsc

_N = 100000
_MAX_DEG = 10
_COUNTS = [10000] + [9000] * _MAX_DEG
_STARTS = np.cumsum([0] + _COUNTS[:-1]).tolist()
_ABASE = [0] + np.cumsum([9000 * d for d in range(1, _MAX_DEG + 1)]).tolist()
_NC, _NS = 2, 16
_NW = _NC * _NS
_DEG_ROWS = 9000


def _divisors(n):
    return sorted({d for i in range(1, int(n ** 0.5) + 1) if n % i == 0
                   for d in (i, n // i)}, reverse=True)


def _pick_chunk(d, gmax):
    for c in _divisors(_DEG_ROWS):
        if c % 8 == 0 and c * d <= gmax:
            return c
    raise ValueError(d)


def _worker_id():
    return lax.axis_index("s") * _NC + lax.axis_index("c")


def _sc_mesh():
    return plsc.VectorSubcoreMesh(core_axis_name="c", subcore_axis_name="s")


def _conv_plan(gmax):
    return [(d, _pick_chunk(d, gmax), _DEG_ROWS // _pick_chunk(d, gmax))
            for d in range(1, _MAX_DEG + 1)]


def _chunk_loop(n_chunks, body):
    wid = _worker_id()
    n_full = n_chunks // _NW
    rem = n_chunks % _NW
    n_mine = n_full + jnp.where(wid < rem, 1, 0)

    def step(k, carry):
        body(wid + k * _NW)
        return carry

    lax.fori_loop(0, n_mine, step, 0)


def _pipelined_loop(n_chunks, issue, consume):
    wid = _worker_id()
    n_full = n_chunks // _NW
    rem = n_chunks % _NW
    m = n_full + jnp.where(wid < rem, 1, 0)

    @pl.when(m > 0)
    def _():
        issue(wid, 0)

    def pair(k2, carry):
        ka = 2 * k2
        kb = ka + 1

        @pl.when(ka < m)
        def _():
            @pl.when(kb < m)
            def _():
                issue(wid + kb * _NW, 1)
            consume(wid + ka * _NW, 0, ka)

        @pl.when(kb < m)
        def _():
            @pl.when(kb + 1 < m)
            def _():
                issue(wid + (kb + 1) * _NW, 0)
            consume(wid + kb * _NW, 1, kb)

        return carry

    lax.fori_loop(0, (m + 1) // 2, pair, 0)
    return m


def _make_conv_sc(D, gmax):
    plan = _conv_plan(gmax)
    g_rows = max(c * d for d, c, _ in plan)
    acc_rows = max(c for d, c, _ in plan if d > 1)

    def body(table, *rest):
        adjs = rest[:_MAX_DEG]
        out = rest[_MAX_DEG]
        rest = rest[_MAX_DEG + 1:]
        idx_v = rest[0:2]
        rows_v = rest[2:4]
        acc_v = rest[4:6]
        gsem = rest[6:8]
        wsem = rest[8:10]

        for d, C, n_chunks in plan:
            G = C * d
            adj = adjs[d - 1]

            def issue(ci, p, d=d, C=C, G=G, adj=adj):
                pltpu.sync_copy(adj.at[pl.ds(ci * G, G)],
                                idx_v[p].at[pl.ds(0, G)])
                pltpu.async_copy(table.at[idx_v[p].at[pl.ds(0, G)]],
                                 rows_v[p].at[pl.ds(0, G)], gsem[p])

            def consume(ci, p, k, d=d, C=C, G=G):
                out0 = (d - 1) * _DEG_ROWS + ci * C
                pltpu.make_async_copy(table.at[idx_v[p].at[pl.ds(0, G)]],
                                      rows_v[p].at[pl.ds(0, G)],
                                      gsem[p]).wait()
                if d == 1:
                    pltpu.sync_copy(rows_v[p].at[pl.ds(0, C)],
                                    out.at[pl.ds(out0, C)])
                    return

                @pl.when(k >= 2)
                def _():
                    pltpu.make_async_copy(acc_v[p].at[pl.ds(0, C)],
                                          out.at[pl.ds(0, C)], wsem[p]).wait()

                def red(r, carry):
                    for c in range(D // 16):
                        sl = pl.ds(c * 16, 16)
                        s = rows_v[p][r * d, sl]
                        for j in range(1, d):
                            s = s + rows_v[p][r * d + j, sl]
                        acc_v[p][r, sl] = s
                    return carry
                lax.fori_loop(0, C, red, 0)
                pltpu.async_copy(acc_v[p].at[pl.ds(0, C)],
                                 out.at[pl.ds(out0, C)], wsem[p])

            m = _pipelined_loop(n_chunks, issue, consume)
            if d > 1:
                for p in range(2):
                    @pl.when(m > p)
                    def _(p=p, C=C):
                        pltpu.make_async_copy(
                            acc_v[p].at[pl.ds(0, C)],
                            out.at[pl.ds(0, C)], wsem[p]).wait()

    return pl.kernel(
        body,
        out_type=jax.ShapeDtypeStruct((_MAX_DEG * _DEG_ROWS, D), jnp.float32),
        mesh=_sc_mesh(),
        compiler_params=pltpu.CompilerParams(use_tc_tiling_on_sc=(D == 128)),
        scratch_types=[
            pltpu.VMEM((g_rows,), jnp.int32),
            pltpu.VMEM((g_rows,), jnp.int32),
            pltpu.VMEM((g_rows, D), jnp.float32),
            pltpu.VMEM((g_rows, D), jnp.float32),
            pltpu.VMEM((acc_rows, D), jnp.float32),
            pltpu.VMEM((acc_rows, D), jnp.float32),
            pltpu.SemaphoreType.DMA,
            pltpu.SemaphoreType.DMA,
            pltpu.SemaphoreType.DMA,
            pltpu.SemaphoreType.DMA,
        ],
    )


def _make_pool_sc(D, gmax):
    cmax = 120 if D == 128 else 360
    plan = [(d, min(c, cmax), _DEG_ROWS // min(c, cmax))
            for d, c, _ in _conv_plan(gmax)]
    c0 = 200
    n0 = _COUNTS[0] // c0
    g_rows = max(max(c * d for d, c, _ in plan), c0)
    acc_rows = max(c for d, c, _ in plan)

    def body(table, *rest):
        adjs = rest[:_MAX_DEG]
        out = rest[_MAX_DEG]
        rest = rest[_MAX_DEG + 1:]
        idx_v = rest[0:2]
        rows_v = rest[2:4]
        acc_v = rest[4:6]
        gsem = rest[6:8]
        wsem = rest[8:10]

        def copy0(ci):
            row0 = ci * c0
            pltpu.sync_copy(table.at[pl.ds(row0, c0)],
                            rows_v[0].at[pl.ds(0, c0)])
            pltpu.sync_copy(rows_v[0].at[pl.ds(0, c0)],
                            out.at[pl.ds(row0, c0)])

        _chunk_loop(n0, copy0)

        for d, C, n_chunks in plan:
            G = C * d
            adj = adjs[d - 1]
            start = _STARTS[d]

            def issue(ci, p, d=d, C=C, G=G, adj=adj, start=start):
                pltpu.sync_copy(adj.at[pl.ds(ci * G, G)],
                                idx_v[p].at[pl.ds(0, G)])
                pltpu.async_copy(table.at[idx_v[p].at[pl.ds(0, G)]],
                                 rows_v[p].at[pl.ds(0, G)], gsem[p])

            def consume(ci, p, k, d=d, C=C, G=G, start=start):
                row0 = start + ci * C

                @pl.when(k >= 2)
                def _():
                    pltpu.make_async_copy(acc_v[p].at[pl.ds(0, C)],
                                          out.at[pl.ds(0, C)], wsem[p]).wait()

                pltpu.sync_copy(table.at[pl.ds(row0, C)],
                                acc_v[p].at[pl.ds(0, C)])
                pltpu.make_async_copy(table.at[idx_v[p].at[pl.ds(0, G)]],
                                      rows_v[p].at[pl.ds(0, G)],
                                      gsem[p]).wait()

                def red(r, carry):
                    for c in range(D // 16):
                        sl = pl.ds(c * 16, 16)
                        m = acc_v[p][r, sl]
                        for j in range(d):
                            m = jnp.maximum(m, rows_v[p][r * d + j, sl])
                        acc_v[p][r, sl] = m
                    return carry
                lax.fori_loop(0, C, red, 0)
                pltpu.async_copy(acc_v[p].at[pl.ds(0, C)],
                                 out.at[pl.ds(row0, C)], wsem[p])

            m = _pipelined_loop(n_chunks, issue, consume)
            for p in range(2):
                @pl.when(m > p)
                def _(p=p, C=C):
                    pltpu.make_async_copy(
                        acc_v[p].at[pl.ds(0, C)],
                        out.at[pl.ds(0, C)], wsem[p]).wait()

    return pl.kernel(
        body,
        out_type=jax.ShapeDtypeStruct((_N, D), jnp.float32),
        mesh=_sc_mesh(),
        compiler_params=pltpu.CompilerParams(use_tc_tiling_on_sc=(D == 128)),
        scratch_types=[
            pltpu.VMEM((g_rows,), jnp.int32),
            pltpu.VMEM((g_rows,), jnp.int32),
            pltpu.VMEM((g_rows, D), jnp.float32),
            pltpu.VMEM((g_rows, D), jnp.float32),
            pltpu.VMEM((acc_rows, D), jnp.float32),
            pltpu.VMEM((acc_rows, D), jnp.float32),
            pltpu.SemaphoreType.DMA,
            pltpu.SemaphoreType.DMA,
            pltpu.SemaphoreType.DMA,
            pltpu.SemaphoreType.DMA,
        ],
    )


_TC_B = 1000
_TC_G = _N // _TC_B


def _deg_of_block(i):
    return jnp.where(i < 10, 0, (i - 10) // 9 + 1)


def _make_gc_tc(din, dout):
    def body(x_ref, rel_ref, wr_ref, ws_ref, b_ref, sc_ref, sh_ref, out_ref):
        z = (jnp.dot(x_ref[...], ws_ref[0],
                     preferred_element_type=jnp.float32)
             + jnp.dot(rel_ref[...], wr_ref[0],
                       preferred_element_type=jnp.float32)
             + b_ref[0])
        out_ref[...] = jnp.tanh(z) * sc_ref[...] + sh_ref[...]

    return pl.pallas_call(
        body,
        grid=(_TC_G,),
        in_specs=[
            pl.BlockSpec((_TC_B, din), lambda i: (i, 0)),
            pl.BlockSpec((_TC_B, din), lambda i: (jnp.maximum(i - 10, 0), 0)),
            pl.BlockSpec((1, din, dout), lambda i: (_deg_of_block(i), 0, 0)),
            pl.BlockSpec((1, din, dout), lambda i: (_deg_of_block(i), 0, 0)),
            pl.BlockSpec((1, 1, dout), lambda i: (_deg_of_block(i), 0, 0)),
            pl.BlockSpec((1, dout), lambda i: (0, 0)),
            pl.BlockSpec((1, dout), lambda i: (0, 0)),
        ],
        out_specs=pl.BlockSpec((_TC_B, dout), lambda i: (i, 0)),
        out_shape=jax.ShapeDtypeStruct((_N, dout), jnp.float32),
    )


def _make_final_tc():
    def body(x_ref, w1_ref, b1_ref, sc_ref, sh_ref, m_ref, wreg_ref, breg_ref,
             out_ref, sum_ref, max_ref):
        i = pl.program_id(0)
        h = jnp.tanh(jnp.dot(x_ref[...], w1_ref[...],
                             preferred_element_type=jnp.float32) + b1_ref[...])
        h = h * sc_ref[...] + sh_ref[...]
        rows = i * _TC_B + jax.lax.broadcasted_iota(jnp.int32, (_TC_B, 1), 0)
        sums, maxs = [], []
        for b in range(3):
            lo = jnp.int32(0) if b == 0 else m_ref[0, b - 1]
            hi = jnp.int32(_N) if b == 2 else m_ref[0, b]
            msk = (rows >= lo) & (rows < hi)
            sums.append(jnp.sum(jnp.where(msk, h, 0.0), axis=0, keepdims=True))
            maxs.append(jnp.max(jnp.where(msk, h, -jnp.inf), axis=0,
                                keepdims=True))
        s = jnp.concatenate(sums, axis=0)
        mx = jnp.concatenate(maxs, axis=0)

        @pl.when(i == 0)
        def _():
            sum_ref[...] = s
            max_ref[...] = mx

        @pl.when(i > 0)
        def _():
            sum_ref[...] = sum_ref[...] + s
            max_ref[...] = jnp.maximum(max_ref[...], mx)

        @pl.when(i == _TC_G - 1)
        def _():
            mol = jnp.tanh(jnp.concatenate([sum_ref[...], max_ref[...]],
                                           axis=1))
            out_ref[...] = (jnp.dot(mol, wreg_ref[...],
                                    preferred_element_type=jnp.float32)
                            + breg_ref[...])

    return pl.pallas_call(
        body,
        grid=(_TC_G,),
        in_specs=[
            pl.BlockSpec((_TC_B, 128), lambda i: (i, 0)),
            pl.BlockSpec((128, 256), lambda i: (0, 0)),
            pl.BlockSpec((1, 256), lambda i: (0, 0)),
            pl.BlockSpec((1, 256), lambda i: (0, 0)),
            pl.BlockSpec((1, 256), lambda i: (0, 0)),
            pl.BlockSpec((1, 2), lambda i: (0, 0)),
            pl.BlockSpec((512, 1), lambda i: (0, 0)),
            pl.BlockSpec((1, 1), lambda i: (0, 0)),
        ],
        out_specs=pl.BlockSpec((3, 1), lambda i: (0, 0)),
        out_shape=jax.ShapeDtypeStruct((3, 1), jnp.float32),
        scratch_shapes=[
            pltpu.VMEM((3, 256), jnp.float32),
            pltpu.VMEM((3, 256), jnp.float32),
        ],
    )


def _bn_affine(p, eps=1e-3):
    scale = p["gamma"] / jnp.sqrt(p["var"] + eps)
    shift = p["beta"] - p["mean"] * scale
    return scale[None, :], shift[None, :]


def _gc_weights(gc, din, dout):
    wr = jnp.stack([jnp.zeros((din, dout), jnp.float32)]
                   + [gc["W"][2 * (d - 1)] for d in range(1, _MAX_DEG + 1)])
    ws = jnp.stack([gc["W"][2 * _MAX_DEG]]
                   + [gc["W"][2 * d - 1] for d in range(1, _MAX_DEG + 1)])
    b = jnp.stack([gc["b"][2 * _MAX_DEG]]
                  + [gc["b"][2 * (d - 1)] + gc["b"][2 * d - 1]
                     for d in range(1, _MAX_DEG + 1)])
    return wr, ws, b[:, None, :]


def kernel(atom_features, params, deg_slice, membership, deg_adj_1, deg_adj_2,
           deg_adj_3, deg_adj_4, deg_adj_5, deg_adj_6, deg_adj_7, deg_adj_8,
           deg_adj_9, deg_adj_10):
    del deg_slice
    adjf = [a.reshape(-1) for a in
            (deg_adj_1, deg_adj_2, deg_adj_3, deg_adj_4, deg_adj_5, deg_adj_6,
             deg_adj_7, deg_adj_8, deg_adj_9, deg_adj_10)]
    bnd = jnp.stack([jnp.sum(membership < 1), jnp.sum(membership < 2)])
    bnd = bnd.astype(jnp.int32)[None, :]
    p = params

    wr1, ws1, b1 = _gc_weights(p["gc1"], 128, 64)
    sc1, sh1 = _bn_affine(p["bn1"])
    wr2, ws2, b2 = _gc_weights(p["gc2"], 64, 128)
    sc2, sh2 = _bn_affine(p["bn2"])
    sc3, sh3 = _bn_affine(p["bn3"])

    rel1 = _make_conv_sc(128, 240)(atom_features, *adjf)
    h1 = _make_gc_tc(128, 64)(atom_features, rel1, wr1, ws1, b1, sc1, sh1)
    hp1 = _make_pool_sc(64, 384)(h1, *adjf)
    rel2 = _make_conv_sc(64, 384)(hp1, *adjf)
    h2 = _make_gc_tc(64, 128)(hp1, rel2, wr2, ws2, b2, sc2, sh2)
    hp2 = _make_pool_sc(128, 240)(h2, *adjf)

    out = _make_final_tc()(
        hp2, p["dense1"]["W"], p["dense1"]["b"][None, :], sc3, sh3,
        bnd, p["regress"]["W"], p["regress"]["b"][None, :])
    return out
